# baseline (speedup 1.0000x reference)
import jax
import jax.numpy as jnp
from jax.experimental import pallas as pl


def _gather_kernel(x_ref, o_ref):
    o_ref[...] = x_ref[:, 0, 5:30]


def kernel(x):
    xf = jnp.reshape(x, (128, 1280, 128))
    y2 = pl.pallas_call(
        _gather_kernel,
        out_shape=jax.ShapeDtypeStruct((128, 25), x.dtype),
        grid=(1,),
        in_specs=[pl.BlockSpec((128, 8, 128), lambda i: (0, 0, 0))],
        out_specs=pl.BlockSpec((128, 25), lambda i: (0, 0)),
    )(xf)
    return y2.reshape(128, 1, 5, 5)


# speedup vs baseline: 55.7725x; 55.7725x over previous
import jax
import jax.numpy as jnp
from jax.experimental import pallas as pl


def _gather_kernel(x_ref, o_ref):
    o_ref[...] = x_ref[:, :, 0, 1:6]


def kernel(x):
    xv = jnp.transpose(x, (0, 1, 3, 2)).reshape(128, 5, 256, 128)
    g = pl.pallas_call(
        _gather_kernel,
        out_shape=jax.ShapeDtypeStruct((128, 5, 5), x.dtype),
        grid=(1,),
        in_specs=[pl.BlockSpec((128, 5, 8, 128), lambda i: (0, 0, 0, 0))],
        out_specs=pl.BlockSpec((128, 5, 5), lambda i: (0, 0, 0)),
    )(xv)
    return jnp.transpose(g, (0, 2, 1)).reshape(128, 1, 5, 5)
